# baseline (speedup 1.0000x reference)
import functools

import jax
import jax.numpy as jnp
from jax import lax
from jax.experimental import pallas as pl
from jax.experimental.pallas import tpu as pltpu
from jax.experimental.pallas import tpu_sc as plsc

N = 10000
NP = 10240
E = 320000
NC = 2
NS = 16
NW = NC * NS
EPT = E // NW

NT = NP // NC
TR = 128
NDR = NT + TR
HSTRIPE = NT // NS

RCH = 125
REGC = 83
DEG_W = 16

DCH = 125
DNCHUNK = E // DCH
DROWS = DNCHUNK // NS
DSTRIPE = NDR // NS


def _part_body(src_ref, dst_ref, srcp_ref, srcq_ref, dstp_ref, cnt_ref,
               sv, dv, sb0, db0, sb1, db1, qb0, qb1, cv):
    c = lax.axis_index("c")
    s = lax.axis_index("s")
    w = c * NS + s
    pltpu.sync_copy(src_ref.at[pl.ds(w * EPT, EPT)], sv)
    pltpu.sync_copy(dst_ref.at[pl.ds(w * EPT, EPT)], dv)
    lanes = lax.iota(jnp.int32, 16)

    def step(i, carry):
        off0, off1 = carry
        base = pl.multiple_of(i * 16, 16)
        s16 = sv[pl.ds(base, 16)]
        d16 = dv[pl.ds(base, 16)]
        m0 = d16 < NT
        m1 = jnp.logical_not(m0)
        mi = jnp.where(m0, 1, 0).astype(jnp.int32)
        cs = plsc.cumsum(mi)
        pos0 = off0 + cs - mi
        pos1 = off1 + lanes - cs + mi
        r0 = [pos0 // RCH, pos0 % RCH]
        r1 = [pos1 // RCH, pos1 % RCH]
        plsc.store_scatter(sb0, r0, s16, mask=m0)
        plsc.store_scatter(qb0, r0, s16 + NP, mask=m0)
        plsc.store_scatter(db0, r0, d16, mask=m0)
        plsc.store_scatter(sb1, r1, s16, mask=m1)
        plsc.store_scatter(qb1, r1, s16 + NP, mask=m1)
        plsc.store_scatter(db1, r1, d16 - NT, mask=m1)
        c0 = jnp.max(cs)
        return off0 + c0, off1 + (16 - c0)

    off0, off1 = lax.fori_loop(0, EPT // 16, step,
                               (jnp.int32(0), jnp.int32(0)))
    zeros = jnp.zeros((16,), jnp.int32)
    trash = NT + lanes
    for k in range(16):
        p0 = off0 + k * 16 + lanes
        r0 = [p0 // RCH, p0 % RCH]
        plsc.store_scatter(sb0, r0, zeros)
        plsc.store_scatter(qb0, r0, zeros)
        plsc.store_scatter(db0, r0, trash)
        p1 = off1 + k * 16 + lanes
        r1 = [p1 // RCH, p1 % RCH]
        plsc.store_scatter(sb1, r1, zeros)
        plsc.store_scatter(qb1, r1, zeros)
        plsc.store_scatter(db1, r1, trash)
    cv[0] = jnp.broadcast_to(off0, (16,)).astype(jnp.int32)
    cv[1] = jnp.broadcast_to(off1, (16,)).astype(jnp.int32)
    sh = lax.shift_right_logical(w, 1)
    sk = jnp.bitwise_and(w, 1)
    pltpu.sync_copy(cv.at[0], cnt_ref.at[0, sh, sk])
    pltpu.sync_copy(cv.at[1], cnt_ref.at[1, sh, sk])
    pltpu.sync_copy(sb0, srcp_ref.at[0, w])
    pltpu.sync_copy(sb1, srcp_ref.at[1, w])
    pltpu.sync_copy(qb0, srcq_ref.at[0, w])
    pltpu.sync_copy(qb1, srcq_ref.at[1, w])
    pltpu.sync_copy(db0, dstp_ref.at[0, w])
    pltpu.sync_copy(db1, dstp_ref.at[1, w])


def _partition(src, dst):
    mesh = plsc.VectorSubcoreMesh(core_axis_name="c", subcore_axis_name="s")
    return pl.kernel(
        _part_body,
        out_type=(
            jax.ShapeDtypeStruct((NC, NW, REGC, RCH), jnp.int32),
            jax.ShapeDtypeStruct((NC, NW, REGC, RCH), jnp.int32),
            jax.ShapeDtypeStruct((NC, NW, REGC, RCH), jnp.int32),
            jax.ShapeDtypeStruct((NC, NS, 2, 16), jnp.int32),
        ),
        mesh=mesh,
        compiler_params=pltpu.CompilerParams(needs_layout_passes=False),
        scratch_types=[
            pltpu.VMEM((EPT,), jnp.int32),
            pltpu.VMEM((EPT,), jnp.int32),
            pltpu.VMEM((REGC, RCH), jnp.int32),
            pltpu.VMEM((REGC, RCH), jnp.int32),
            pltpu.VMEM((REGC, RCH), jnp.int32),
            pltpu.VMEM((REGC, RCH), jnp.int32),
            pltpu.VMEM((REGC, RCH), jnp.int32),
            pltpu.VMEM((REGC, RCH), jnp.int32),
            pltpu.VMEM((2, 16), jnp.int32),
        ],
    )(src, dst)


PMAX = (REGC - 1) // 2


def _seg_loop(u_ref, idx_s, idx_d, b0, b1, acc, s0, s1, pairs):
    nche = 2 * pairs
    pltpu.async_copy(u_ref.at[idx_s.at[0]], b0, s0)

    def step(jj, carry):
        j0 = 2 * jj

        @pl.when(j0 < nche)
        def _():
            j1 = j0 + 1
            j2 = jnp.minimum(j0 + 2, nche - 1)
            pltpu.make_async_copy(u_ref.at[idx_s.at[j0]], b0, s0).wait()
            pltpu.async_copy(u_ref.at[idx_s.at[j1]], b1, s1)
            pltpu.sync_copy(b0, acc.at[idx_d.at[j0]], add=True)
            pltpu.make_async_copy(u_ref.at[idx_s.at[j1]], b1, s1).wait()
            pltpu.async_copy(u_ref.at[idx_s.at[j2]], b0, s0)
            pltpu.sync_copy(b1, acc.at[idx_d.at[j1]], add=True)

        return carry

    lax.fori_loop(0, PMAX, step, 0)
    pltpu.make_async_copy(u_ref.at[idx_s.at[0]], b0, s0).wait()


def _pairs_of(cv, k):
    cnt = cv[k][0]
    nch = (cnt + (RCH - 1)) // RCH
    return lax.shift_right_logical(nch + 1, 1)


def _seg1_body(u_ref, srcp_ref, dstp_ref, cnt_ref, out_ref,
               idx_s, idx_d, b0, b1, cv, acc, s0, s1):
    c = lax.axis_index("c")
    s = lax.axis_index("s")
    pltpu.sync_copy(u_ref.at[pl.ds(c * NT + s * HSTRIPE, HSTRIPE)],
                    acc.at[pl.ds(s * HSTRIPE, HSTRIPE)])
    pltpu.sync_copy(cnt_ref.at[c, s], cv)
    plsc.subcore_barrier()
    for k in range(2):
        r = 2 * s + k
        pltpu.sync_copy(srcp_ref.at[c, r], idx_s)
        pltpu.sync_copy(dstp_ref.at[c, r], idx_d)
        _seg_loop(u_ref, idx_s, idx_d, b0, b1, acc, s0, s1, _pairs_of(cv, k))
    plsc.subcore_barrier()
    pltpu.sync_copy(acc.at[pl.ds(s * HSTRIPE, HSTRIPE)],
                    out_ref.at[c, pl.ds(s * HSTRIPE, HSTRIPE)])


def _segment_sum1(u1, srcp, dstp, cnt):
    mesh = plsc.VectorSubcoreMesh(core_axis_name="c", subcore_axis_name="s")
    return pl.kernel(
        _seg1_body,
        out_type=jax.ShapeDtypeStruct((NC, NT, 128), jnp.float32),
        mesh=mesh,
        scratch_types=[
            pltpu.VMEM((REGC, RCH), jnp.int32),
            pltpu.VMEM((REGC, RCH), jnp.int32),
            pltpu.VMEM((RCH, 128), jnp.float32),
            pltpu.VMEM((RCH, 128), jnp.float32),
            pltpu.VMEM((2, 16), jnp.int32),
            pltpu.VMEM_SHARED((NDR, 128), jnp.float32),
            pltpu.SemaphoreType.DMA,
            pltpu.SemaphoreType.DMA,
        ],
    )(u1, srcp, dstp, cnt)


def _seg2_body(h, u_ref, srcp_ref, srcq_ref, dstp_ref, cnt_ref, out_ref,
               idx_s, idx_d, b0, b1, cv, acc, s0, s1):
    c = lax.axis_index("c")
    s = lax.axis_index("s")
    pltpu.sync_copy(u_ref.at[pl.ds(c * NP + h * NT + s * HSTRIPE, HSTRIPE)],
                    acc.at[pl.ds(s * HSTRIPE, HSTRIPE)])
    pltpu.sync_copy(cnt_ref.at[h, s], cv)
    plsc.subcore_barrier()
    for k in range(2):
        r = 2 * s + k

        @pl.when(c == 0)
        def _():
            pltpu.sync_copy(srcp_ref.at[h, r], idx_s)

        @pl.when(c == 1)
        def _():
            pltpu.sync_copy(srcq_ref.at[h, r], idx_s)

        pltpu.sync_copy(dstp_ref.at[h, r], idx_d)
        _seg_loop(u_ref, idx_s, idx_d, b0, b1, acc, s0, s1, _pairs_of(cv, k))
    plsc.subcore_barrier()
    pltpu.sync_copy(acc.at[pl.ds(s * HSTRIPE, HSTRIPE)],
                    out_ref.at[c, pl.ds(s * HSTRIPE, HSTRIPE)])


def _segment_sum2(u_flat, srcp, srcq, dstp, cnt, h):
    mesh = plsc.VectorSubcoreMesh(core_axis_name="c", subcore_axis_name="s")
    return pl.kernel(
        functools.partial(_seg2_body, h),
        out_type=jax.ShapeDtypeStruct((NC, NT, 128), jnp.float32),
        mesh=mesh,
        scratch_types=[
            pltpu.VMEM((REGC, RCH), jnp.int32),
            pltpu.VMEM((REGC, RCH), jnp.int32),
            pltpu.VMEM((RCH, 128), jnp.float32),
            pltpu.VMEM((RCH, 128), jnp.float32),
            pltpu.VMEM((2, 16), jnp.int32),
            pltpu.VMEM_SHARED((NDR, 128), jnp.float32),
            pltpu.SemaphoreType.DMA,
            pltpu.SemaphoreType.DMA,
        ],
    )(u_flat, srcp, srcq, dstp, cnt)


def _deg_body(dstd_ref, out_ref, idx_v, ones_v, acc):
    c = lax.axis_index("c")
    s = lax.axis_index("s")
    pltpu.sync_copy(dstd_ref.at[c, pl.ds(s * DROWS, DROWS)], idx_v)
    for i in range(DCH):
        ones_v[i] = jnp.full((DEG_W,), 1.0, jnp.float32)
    base = s * DSTRIPE
    pltpu.sync_copy(ones_v, acc.at[pl.ds(base, DCH)])
    pltpu.sync_copy(ones_v, acc.at[pl.ds(base + DCH, DCH)])
    pltpu.sync_copy(ones_v.at[pl.ds(0, DSTRIPE - 2 * DCH)],
                    acc.at[pl.ds(base + 2 * DCH, DSTRIPE - 2 * DCH)])
    plsc.subcore_barrier()

    def step(j, carry):
        pltpu.sync_copy(ones_v, acc.at[idx_v.at[j]], add=True)
        return carry

    lax.fori_loop(0, DROWS, step, 0)
    plsc.subcore_barrier()
    pltpu.sync_copy(acc.at[pl.ds(base, DSTRIPE)],
                    out_ref.at[c, pl.ds(base, DSTRIPE)])


def _degree(dstd):
    mesh = plsc.VectorSubcoreMesh(core_axis_name="c", subcore_axis_name="s")
    return pl.kernel(
        _deg_body,
        out_type=jax.ShapeDtypeStruct((NC, NDR, DEG_W), jnp.float32),
        mesh=mesh,
        scratch_types=[
            pltpu.VMEM((DROWS, DCH), jnp.int32),
            pltpu.VMEM((DCH, DEG_W), jnp.float32),
            pltpu.VMEM_SHARED((NDR, DEG_W), jnp.float32),
        ],
    )(dstd)


def _leaky(v):
    return jnp.where(v >= 0, v, 0.01 * v)


def _tc1_body(deg_ref, x_ref, dinv_ref, u_ref):
    deg = jnp.concatenate(
        [deg_ref[0, :NT, 0:1], deg_ref[1, :NT, 0:1]], axis=0)
    row = lax.broadcasted_iota(jnp.int32, (NP, 1), 0)
    dinv = jnp.where(row < N, lax.rsqrt(deg), 0.0)
    dinv_ref[...] = dinv
    u_ref[...] = x_ref[...] * dinv


def _tc1(deg, xpad):
    return pl.pallas_call(
        _tc1_body,
        out_shape=(
            jax.ShapeDtypeStruct((NP, 1), jnp.float32),
            jax.ShapeDtypeStruct((NP, 128), jnp.float32),
        ),
    )(deg, xpad)


_B2 = 640
_G2 = NP // _B2


def _tc2_body(pacc_ref, dinv_ref, w_ref, b_ref, u2_ref):
    dinv = dinv_ref[...]
    p = pacc_ref[...] * dinv
    h = jnp.dot(p, w_ref[...], preferred_element_type=jnp.float32) + b_ref[...]
    u2 = _leaky(h) * dinv
    u2_ref[0] = u2[:, :128]
    u2_ref[1] = u2[:, 128:]


def _tc2(pacc, dinv, w1, b1):
    return pl.pallas_call(
        _tc2_body,
        grid=(_G2,),
        in_specs=[
            pl.BlockSpec((_B2, 128), lambda i: (i, 0)),
            pl.BlockSpec((_B2, 1), lambda i: (i, 0)),
            pl.BlockSpec((128, 256), lambda i: (0, 0)),
            pl.BlockSpec((1, 256), lambda i: (0, 0)),
        ],
        out_specs=pl.BlockSpec((NC, _B2, 128), lambda i: (0, i, 0)),
        out_shape=jax.ShapeDtypeStruct((NC, NP, 128), jnp.float32),
    )(pacc, dinv, w1, b1)


_B3 = 1000
_G3 = N // _B3


def _tc3_body(pacc_ref, dinv_ref, w_ref, b_ref, wm1_ref, bm1_ref, wm2_ref,
              bm2_ref, out_ref, acc_ref):
    i = pl.program_id(0)
    p = jnp.concatenate([pacc_ref[0], pacc_ref[1]], axis=1) * dinv_ref[...]
    h = _leaky(jnp.dot(p, w_ref[...], preferred_element_type=jnp.float32)
               + b_ref[...])
    part = jnp.sum(h, axis=0, keepdims=True)

    @pl.when(i == 0)
    def _():
        acc_ref[...] = part

    @pl.when(i > 0)
    def _():
        acc_ref[...] += part

    @pl.when(i == _G3 - 1)
    def _():
        g = acc_ref[...] * (1.0 / N)
        m = _leaky(jnp.dot(g, wm1_ref[...], preferred_element_type=jnp.float32)
                   + bm1_ref[...])
        out_ref[...] = _leaky(
            jnp.dot(m, wm2_ref[...], preferred_element_type=jnp.float32)
            + bm2_ref[...])


def _tc3(pacc, dinv, w2, b2, wm1, bm1, wm2, bm2):
    return pl.pallas_call(
        _tc3_body,
        grid=(_G3,),
        in_specs=[
            pl.BlockSpec((NC, _B3, 128), lambda i: (0, i, 0)),
            pl.BlockSpec((_B3, 1), lambda i: (i, 0)),
            pl.BlockSpec((256, 256), lambda i: (0, 0)),
            pl.BlockSpec((1, 256), lambda i: (0, 0)),
            pl.BlockSpec((256, 512), lambda i: (0, 0)),
            pl.BlockSpec((1, 512), lambda i: (0, 0)),
            pl.BlockSpec((512, 128), lambda i: (0, 0)),
            pl.BlockSpec((1, 128), lambda i: (0, 0)),
        ],
        out_specs=pl.BlockSpec((1, 128), lambda i: (0, 0)),
        out_shape=jax.ShapeDtypeStruct((1, 128), jnp.float32),
        scratch_shapes=[pltpu.VMEM((1, 256), jnp.float32)],
    )(pacc, dinv, w2, b2, wm1, bm1, wm2, bm2)


def kernel(x, edge_index, W1, b1, W2, b2, Wm1, bm1, Wm2, bm2):
    src = edge_index[0]
    dst = edge_index[1]
    xpad = jnp.pad(x, ((0, NP - N), (0, 0)))
    dstd = jnp.stack(
        [jnp.where(dst < NT, dst, NT + (dst & (TR - 1))),
         jnp.where(dst >= NT, dst - NT, NT + (dst & (TR - 1)))],
    ).reshape(NC, DNCHUNK, DCH)

    srcp, srcq, dstp, cnt = _partition(src, dst)
    deg = _degree(dstd)
    dinv, u1 = _tc1(deg, xpad)

    p1 = _segment_sum1(u1, srcp, dstp, cnt).reshape(NP, 128)
    u2 = _tc2(p1, dinv, W1, b1.reshape(1, 256))

    u2_flat = u2.reshape(NC * NP, 128)
    p2 = jnp.concatenate(
        [_segment_sum2(u2_flat, srcp, srcq, dstp, cnt, 0),
         _segment_sum2(u2_flat, srcp, srcq, dstp, cnt, 1)], axis=1)
    return _tc3(p2, dinv, W2, b2.reshape(1, 256), Wm1, bm1.reshape(1, 512),
                Wm2, bm2.reshape(1, 128))


# speedup vs baseline: 2.4915x; 2.4915x over previous
import functools

import jax
import jax.numpy as jnp
from jax import lax
from jax.experimental import pallas as pl
from jax.experimental.pallas import tpu as pltpu
from jax.experimental.pallas import tpu_sc as plsc

N = 10000
NP = 10240
E = 320000
NC = 2
NS = 16
CHUNK = 125
NCHUNK = E // CHUNK
SEG_ROWS = NCHUNK // NS
DEG_ROWS = NCHUNK // (NC * NS)
RPT = NP // NS
DEG_W = 16


def _seg_loop(u_ref, idx_s, idx_d, b0, b1, acc, s0, s1, nrows):
    pltpu.async_copy(u_ref.at[idx_s.at[0]], b0, s0)

    def step(jj, carry):
        j0 = 2 * jj
        j1 = j0 + 1
        j2 = jnp.minimum(j0 + 2, nrows - 1)
        pltpu.make_async_copy(u_ref.at[idx_s.at[j0]], b0, s0).wait()
        pltpu.async_copy(u_ref.at[idx_s.at[j1]], b1, s1)
        pltpu.sync_copy(b0, acc.at[idx_d.at[j0]], add=True)
        pltpu.make_async_copy(u_ref.at[idx_s.at[j1]], b1, s1).wait()
        pltpu.async_copy(u_ref.at[idx_s.at[j2]], b0, s0)
        pltpu.sync_copy(b1, acc.at[idx_d.at[j1]], add=True)
        return carry

    lax.fori_loop(0, nrows // 2, step, 0)
    pltpu.make_async_copy(u_ref.at[idx_s.at[nrows - 1]], b0, s0).wait()


NT = NP // NC
TR = 128
NDR = NT + TR
HSTRIPE = NT // NS


def _seg1_body(h, u_ref, src_ref, dstd_ref, out_ref, idx_s, idx_d, b0, b1,
               acc, s0, s1):
    c = lax.axis_index("c")
    s = lax.axis_index("s")
    w = c * NS + s
    pltpu.sync_copy(src_ref.at[pl.ds(w * DEG_ROWS, DEG_ROWS)], idx_s)
    pltpu.sync_copy(dstd_ref.at[h, pl.ds(w * DEG_ROWS, DEG_ROWS)], idx_d)
    pltpu.sync_copy(u_ref.at[pl.ds(h * NT + s * HSTRIPE, HSTRIPE)],
                    acc.at[pl.ds(s * HSTRIPE, HSTRIPE)])
    plsc.subcore_barrier()
    _seg_loop(u_ref, idx_s, idx_d, b0, b1, acc, s0, s1, DEG_ROWS)
    plsc.subcore_barrier()
    pltpu.sync_copy(acc.at[pl.ds(s * HSTRIPE, HSTRIPE)],
                    out_ref.at[c, pl.ds(s * HSTRIPE, HSTRIPE)])


def _segment_sum1(u1, src_chunks, dstd, h):
    mesh = plsc.VectorSubcoreMesh(core_axis_name="c", subcore_axis_name="s")
    return pl.kernel(
        functools.partial(_seg1_body, h),
        out_type=jax.ShapeDtypeStruct((NC, NT, 128), jnp.float32),
        mesh=mesh,
        scratch_types=[
            pltpu.VMEM((DEG_ROWS, CHUNK), jnp.int32),
            pltpu.VMEM((DEG_ROWS, CHUNK), jnp.int32),
            pltpu.VMEM((CHUNK, 128), jnp.float32),
            pltpu.VMEM((CHUNK, 128), jnp.float32),
            pltpu.VMEM_SHARED((NDR, 128), jnp.float32),
            pltpu.SemaphoreType.DMA,
            pltpu.SemaphoreType.DMA,
        ],
    )(u1, src_chunks, dstd)


def _seg2_body(h, u_ref, srcg_ref, dstd_ref, out_ref, idx_s, idx_d, b0, b1,
               acc, s0, s1):
    c = lax.axis_index("c")
    s = lax.axis_index("s")
    pltpu.sync_copy(srcg_ref.at[c, pl.ds(s * SEG_ROWS, SEG_ROWS)], idx_s)
    pltpu.sync_copy(dstd_ref.at[h, pl.ds(s * SEG_ROWS, SEG_ROWS)], idx_d)
    pltpu.sync_copy(u_ref.at[pl.ds(c * NP + h * NT + s * HSTRIPE, HSTRIPE)],
                    acc.at[pl.ds(s * HSTRIPE, HSTRIPE)])
    plsc.subcore_barrier()
    _seg_loop(u_ref, idx_s, idx_d, b0, b1, acc, s0, s1, SEG_ROWS)
    plsc.subcore_barrier()
    pltpu.sync_copy(acc.at[pl.ds(s * HSTRIPE, HSTRIPE)],
                    out_ref.at[c, pl.ds(s * HSTRIPE, HSTRIPE)])


def _segment_sum2(u_flat, srcg, dstd, h):
    mesh = plsc.VectorSubcoreMesh(core_axis_name="c", subcore_axis_name="s")
    return pl.kernel(
        functools.partial(_seg2_body, h),
        out_type=jax.ShapeDtypeStruct((NC, NT, 128), jnp.float32),
        mesh=mesh,
        scratch_types=[
            pltpu.VMEM((SEG_ROWS, CHUNK), jnp.int32),
            pltpu.VMEM((SEG_ROWS, CHUNK), jnp.int32),
            pltpu.VMEM((CHUNK, 128), jnp.float32),
            pltpu.VMEM((CHUNK, 128), jnp.float32),
            pltpu.VMEM_SHARED((NDR, 128), jnp.float32),
            pltpu.SemaphoreType.DMA,
            pltpu.SemaphoreType.DMA,
        ],
    )(u_flat, srcg, dstd)


DSTRIPE = NDR // NS


def _deg_body(dstd_ref, out_ref, idx_v, ones_v, acc):
    c = lax.axis_index("c")
    s = lax.axis_index("s")
    pltpu.sync_copy(dstd_ref.at[c, pl.ds(s * SEG_ROWS, SEG_ROWS)], idx_v)
    for i in range(CHUNK):
        ones_v[i] = jnp.full((DEG_W,), 1.0, jnp.float32)
    base = s * DSTRIPE
    pltpu.sync_copy(ones_v, acc.at[pl.ds(base, CHUNK)])
    pltpu.sync_copy(ones_v, acc.at[pl.ds(base + CHUNK, CHUNK)])
    pltpu.sync_copy(ones_v.at[pl.ds(0, DSTRIPE - 2 * CHUNK)],
                    acc.at[pl.ds(base + 2 * CHUNK, DSTRIPE - 2 * CHUNK)])
    plsc.subcore_barrier()

    def step(j, carry):
        pltpu.sync_copy(ones_v, acc.at[idx_v.at[j]], add=True)
        return carry

    lax.fori_loop(0, SEG_ROWS, step, 0)
    plsc.subcore_barrier()
    pltpu.sync_copy(acc.at[pl.ds(base, DSTRIPE)],
                    out_ref.at[c, pl.ds(base, DSTRIPE)])


def _degree(dstd):
    mesh = plsc.VectorSubcoreMesh(core_axis_name="c", subcore_axis_name="s")
    return pl.kernel(
        _deg_body,
        out_type=jax.ShapeDtypeStruct((NC, NDR, DEG_W), jnp.float32),
        mesh=mesh,
        scratch_types=[
            pltpu.VMEM((SEG_ROWS, CHUNK), jnp.int32),
            pltpu.VMEM((CHUNK, DEG_W), jnp.float32),
            pltpu.VMEM_SHARED((NDR, DEG_W), jnp.float32),
        ],
    )(dstd)


def _leaky(v):
    return jnp.where(v >= 0, v, 0.01 * v)


def _tc1_body(deg_ref, x_ref, dinv_ref, u_ref):
    deg = jnp.concatenate(
        [deg_ref[0, :NT, 0:1], deg_ref[1, :NT, 0:1]], axis=0)
    row = lax.broadcasted_iota(jnp.int32, (NP, 1), 0)
    dinv = jnp.where(row < N, lax.rsqrt(deg), 0.0)
    dinv_ref[...] = dinv
    u_ref[...] = x_ref[...] * dinv


def _tc1(deg, xpad):
    return pl.pallas_call(
        _tc1_body,
        out_shape=(
            jax.ShapeDtypeStruct((NP, 1), jnp.float32),
            jax.ShapeDtypeStruct((NP, 128), jnp.float32),
        ),
    )(deg, xpad)


_B2 = 640
_G2 = NP // _B2


def _tc2_body(pacc_ref, u1_ref, dinv_ref, w_ref, b_ref, u2_ref):
    dinv = dinv_ref[...]
    p = (pacc_ref[0] + pacc_ref[1] - u1_ref[...]) * dinv
    h = jnp.dot(p, w_ref[...], preferred_element_type=jnp.float32) + b_ref[...]
    u2 = _leaky(h) * dinv
    u2_ref[0] = u2[:, :128]
    u2_ref[1] = u2[:, 128:]


def _tc2(pacc, u1, dinv, w1, b1):
    return pl.pallas_call(
        _tc2_body,
        grid=(_G2,),
        in_specs=[
            pl.BlockSpec((NC, _B2, 128), lambda i: (0, i, 0)),
            pl.BlockSpec((_B2, 128), lambda i: (i, 0)),
            pl.BlockSpec((_B2, 1), lambda i: (i, 0)),
            pl.BlockSpec((128, 256), lambda i: (0, 0)),
            pl.BlockSpec((1, 256), lambda i: (0, 0)),
        ],
        out_specs=pl.BlockSpec((NC, _B2, 128), lambda i: (0, i, 0)),
        out_shape=jax.ShapeDtypeStruct((NC, NP, 128), jnp.float32),
    )(pacc, u1, dinv, w1, b1)


_B3 = 1000
_G3 = N // _B3


def _tc3_body(pacc_ref, dinv_ref, w_ref, b_ref, wm1_ref, bm1_ref, wm2_ref,
              bm2_ref, out_ref, acc_ref):
    i = pl.program_id(0)
    p = jnp.concatenate([pacc_ref[0], pacc_ref[1]], axis=1) * dinv_ref[...]
    h = _leaky(jnp.dot(p, w_ref[...], preferred_element_type=jnp.float32)
               + b_ref[...])
    part = jnp.sum(h, axis=0, keepdims=True)

    @pl.when(i == 0)
    def _():
        acc_ref[...] = part

    @pl.when(i > 0)
    def _():
        acc_ref[...] += part

    @pl.when(i == _G3 - 1)
    def _():
        g = acc_ref[...] * (1.0 / N)
        m = _leaky(jnp.dot(g, wm1_ref[...], preferred_element_type=jnp.float32)
                   + bm1_ref[...])
        out_ref[...] = _leaky(
            jnp.dot(m, wm2_ref[...], preferred_element_type=jnp.float32)
            + bm2_ref[...])


def _tc3(pacc, dinv, w2, b2, wm1, bm1, wm2, bm2):
    return pl.pallas_call(
        _tc3_body,
        grid=(_G3,),
        in_specs=[
            pl.BlockSpec((NC, _B3, 128), lambda i: (0, i, 0)),
            pl.BlockSpec((_B3, 1), lambda i: (i, 0)),
            pl.BlockSpec((256, 256), lambda i: (0, 0)),
            pl.BlockSpec((1, 256), lambda i: (0, 0)),
            pl.BlockSpec((256, 512), lambda i: (0, 0)),
            pl.BlockSpec((1, 512), lambda i: (0, 0)),
            pl.BlockSpec((512, 128), lambda i: (0, 0)),
            pl.BlockSpec((1, 128), lambda i: (0, 0)),
        ],
        out_specs=pl.BlockSpec((1, 128), lambda i: (0, 0)),
        out_shape=jax.ShapeDtypeStruct((1, 128), jnp.float32),
        scratch_shapes=[pltpu.VMEM((1, 256), jnp.float32)],
    )(pacc, dinv, w2, b2, wm1, bm1, wm2, bm2)


def kernel(x, edge_index, W1, b1, W2, b2, Wm1, bm1, Wm2, bm2):
    src = edge_index[0]
    dst = edge_index[1]
    src_chunks = src.reshape(NCHUNK, CHUNK)
    dst_chunks = dst.reshape(NCHUNK, CHUNK)
    srcg = jnp.stack([src, src + NP]).reshape(NC, NCHUNK, CHUNK)
    xpad = jnp.pad(x, ((0, NP - N), (0, 0)))
    dstd = jnp.stack(
        [jnp.where(dst < NT, dst, NT + (dst & (TR - 1))),
         jnp.where(dst >= NT, dst - NT, NT + (dst & (TR - 1)))],
    ).reshape(NC, NCHUNK, CHUNK)

    deg = _degree(dstd)
    dinv, u1 = _tc1(deg, xpad)

    p1 = jnp.concatenate(
        [_segment_sum1(u1, src_chunks, dstd, 0),
         _segment_sum1(u1, src_chunks, dstd, 1)], axis=1)
    u2 = _tc2(p1, u1, dinv, W1, b1.reshape(1, 256))

    u2_flat = u2.reshape(NC * NP, 128)
    p2 = jnp.concatenate(
        [_segment_sum2(u2_flat, srcg, dstd, 0),
         _segment_sum2(u2_flat, srcg, dstd, 1)], axis=1)
    return _tc3(p2, dinv, W2, b2.reshape(1, 256), Wm1, bm1.reshape(1, 512),
                Wm2, bm2.reshape(1, 128))


# speedup vs baseline: 2.6218x; 1.0523x over previous
import functools

import jax
import jax.numpy as jnp
from jax import lax
from jax.experimental import pallas as pl
from jax.experimental.pallas import tpu as pltpu
from jax.experimental.pallas import tpu_sc as plsc

N = 10000
NP = 10240
E = 320000
NC = 2
NS = 16
CHUNK = 125
NCHUNK = E // CHUNK
SEG_ROWS = NCHUNK // NS
DEG_ROWS = NCHUNK // (NC * NS)
RPT = NP // NS
DEG_W = 16


def _seg_loop(u_ref, idx_s, idx_d, b0, b1, acc, s0, s1, nrows):
    pltpu.async_copy(u_ref.at[idx_s.at[0]], b0, s0)

    def step(jj, carry):
        j0 = 2 * jj
        j1 = j0 + 1
        j2 = jnp.minimum(j0 + 2, nrows - 1)
        pltpu.make_async_copy(u_ref.at[idx_s.at[j0]], b0, s0).wait()
        pltpu.async_copy(u_ref.at[idx_s.at[j1]], b1, s1)
        pltpu.sync_copy(b0, acc.at[idx_d.at[j0]], add=True)
        pltpu.make_async_copy(u_ref.at[idx_s.at[j1]], b1, s1).wait()
        pltpu.async_copy(u_ref.at[idx_s.at[j2]], b0, s0)
        pltpu.sync_copy(b1, acc.at[idx_d.at[j1]], add=True)
        return carry

    lax.fori_loop(0, nrows // 2, step, 0)
    pltpu.make_async_copy(u_ref.at[idx_s.at[nrows - 1]], b0, s0).wait()


NT = NP // NC
TR = 128
NDR = NT + TR
HSTRIPE = NT // NS


def _seg1_body(u_ref, src_ref, dstd_ref, out_ref, idx_s, idx_d, b0, b1,
               acc, s0, s1):
    c = lax.axis_index("c")
    s = lax.axis_index("s")
    w = c * NS + s
    pltpu.sync_copy(src_ref.at[pl.ds(w * DEG_ROWS, DEG_ROWS)], idx_s)
    for h in range(2):
        pltpu.sync_copy(dstd_ref.at[h, pl.ds(w * DEG_ROWS, DEG_ROWS)], idx_d)
        pltpu.sync_copy(u_ref.at[pl.ds(h * NT + s * HSTRIPE, HSTRIPE)],
                        acc.at[pl.ds(s * HSTRIPE, HSTRIPE)])
        plsc.subcore_barrier()
        _seg_loop(u_ref, idx_s, idx_d, b0, b1, acc, s0, s1, DEG_ROWS)
        plsc.subcore_barrier()
        pltpu.sync_copy(acc.at[pl.ds(s * HSTRIPE, HSTRIPE)],
                        out_ref.at[c, h, pl.ds(s * HSTRIPE, HSTRIPE)])


def _segment_sum1(u1, src_chunks, dstd):
    mesh = plsc.VectorSubcoreMesh(core_axis_name="c", subcore_axis_name="s")
    return pl.kernel(
        _seg1_body,
        out_type=jax.ShapeDtypeStruct((NC, 2, NT, 128), jnp.float32),
        mesh=mesh,
        scratch_types=[
            pltpu.VMEM((DEG_ROWS, CHUNK), jnp.int32),
            pltpu.VMEM((DEG_ROWS, CHUNK), jnp.int32),
            pltpu.VMEM((CHUNK, 128), jnp.float32),
            pltpu.VMEM((CHUNK, 128), jnp.float32),
            pltpu.VMEM_SHARED((NDR, 128), jnp.float32),
            pltpu.SemaphoreType.DMA,
            pltpu.SemaphoreType.DMA,
        ],
    )(u1, src_chunks, dstd)


def _seg2_body(u_ref, srcg_ref, dstd_ref, out_ref, idx_s, idx_d, b0, b1,
               acc, s0, s1):
    c = lax.axis_index("c")
    s = lax.axis_index("s")
    pltpu.sync_copy(srcg_ref.at[c, pl.ds(s * SEG_ROWS, SEG_ROWS)], idx_s)
    for h in range(2):
        pltpu.sync_copy(dstd_ref.at[h, pl.ds(s * SEG_ROWS, SEG_ROWS)], idx_d)
        pltpu.sync_copy(
            u_ref.at[pl.ds(c * NP + h * NT + s * HSTRIPE, HSTRIPE)],
            acc.at[pl.ds(s * HSTRIPE, HSTRIPE)])
        plsc.subcore_barrier()
        _seg_loop(u_ref, idx_s, idx_d, b0, b1, acc, s0, s1, SEG_ROWS)
        plsc.subcore_barrier()
        pltpu.sync_copy(acc.at[pl.ds(s * HSTRIPE, HSTRIPE)],
                        out_ref.at[c, h, pl.ds(s * HSTRIPE, HSTRIPE)])


def _segment_sum2(u_flat, srcg, dstd):
    mesh = plsc.VectorSubcoreMesh(core_axis_name="c", subcore_axis_name="s")
    return pl.kernel(
        _seg2_body,
        out_type=jax.ShapeDtypeStruct((NC, 2, NT, 128), jnp.float32),
        mesh=mesh,
        scratch_types=[
            pltpu.VMEM((SEG_ROWS, CHUNK), jnp.int32),
            pltpu.VMEM((SEG_ROWS, CHUNK), jnp.int32),
            pltpu.VMEM((CHUNK, 128), jnp.float32),
            pltpu.VMEM((CHUNK, 128), jnp.float32),
            pltpu.VMEM_SHARED((NDR, 128), jnp.float32),
            pltpu.SemaphoreType.DMA,
            pltpu.SemaphoreType.DMA,
        ],
    )(u_flat, srcg, dstd)


DSTRIPE = NDR // NS


def _deg_body(dstd_ref, out_ref, idx_v, ones_v, acc):
    c = lax.axis_index("c")
    s = lax.axis_index("s")
    pltpu.sync_copy(dstd_ref.at[c, pl.ds(s * SEG_ROWS, SEG_ROWS)], idx_v)
    for i in range(CHUNK):
        ones_v[i] = jnp.full((DEG_W,), 1.0, jnp.float32)
    base = s * DSTRIPE
    pltpu.sync_copy(ones_v, acc.at[pl.ds(base, CHUNK)])
    pltpu.sync_copy(ones_v, acc.at[pl.ds(base + CHUNK, CHUNK)])
    pltpu.sync_copy(ones_v.at[pl.ds(0, DSTRIPE - 2 * CHUNK)],
                    acc.at[pl.ds(base + 2 * CHUNK, DSTRIPE - 2 * CHUNK)])
    plsc.subcore_barrier()

    def step(j, carry):
        pltpu.sync_copy(ones_v, acc.at[idx_v.at[j]], add=True)
        return carry

    lax.fori_loop(0, SEG_ROWS, step, 0)
    plsc.subcore_barrier()
    pltpu.sync_copy(acc.at[pl.ds(base, DSTRIPE)],
                    out_ref.at[c, pl.ds(base, DSTRIPE)])


def _degree(dstd):
    mesh = plsc.VectorSubcoreMesh(core_axis_name="c", subcore_axis_name="s")
    return pl.kernel(
        _deg_body,
        out_type=jax.ShapeDtypeStruct((NC, NDR, DEG_W), jnp.float32),
        mesh=mesh,
        scratch_types=[
            pltpu.VMEM((SEG_ROWS, CHUNK), jnp.int32),
            pltpu.VMEM((CHUNK, DEG_W), jnp.float32),
            pltpu.VMEM_SHARED((NDR, DEG_W), jnp.float32),
        ],
    )(dstd)


def _leaky(v):
    return jnp.where(v >= 0, v, 0.01 * v)


def _tc1_body(deg_ref, x_ref, dinv_ref, u_ref):
    deg = jnp.concatenate(
        [deg_ref[0, :NT, 0:1], deg_ref[1, :NT, 0:1]], axis=0)
    row = lax.broadcasted_iota(jnp.int32, (NP, 1), 0)
    dinv = jnp.where(row < N, lax.rsqrt(deg), 0.0)
    dinv_ref[...] = dinv
    u_ref[...] = x_ref[...] * dinv


def _tc1(deg, xpad):
    return pl.pallas_call(
        _tc1_body,
        out_shape=(
            jax.ShapeDtypeStruct((NP, 1), jnp.float32),
            jax.ShapeDtypeStruct((NP, 128), jnp.float32),
        ),
    )(deg, xpad)


_B2 = 640
_G2 = NP // _B2


def _tc2_body(pacc_ref, u1_ref, dinv_ref, w_ref, b_ref, u2_ref):
    dinv = dinv_ref[...]
    p = (pacc_ref[0] + pacc_ref[1] - u1_ref[...]) * dinv
    h = jnp.dot(p, w_ref[...], preferred_element_type=jnp.float32) + b_ref[...]
    u2 = _leaky(h) * dinv
    u2_ref[0] = u2[:, :128]
    u2_ref[1] = u2[:, 128:]


def _tc2(pacc, u1, dinv, w1, b1):
    return pl.pallas_call(
        _tc2_body,
        grid=(_G2,),
        in_specs=[
            pl.BlockSpec((NC, _B2, 128), lambda i: (0, i, 0)),
            pl.BlockSpec((_B2, 128), lambda i: (i, 0)),
            pl.BlockSpec((_B2, 1), lambda i: (i, 0)),
            pl.BlockSpec((128, 256), lambda i: (0, 0)),
            pl.BlockSpec((1, 256), lambda i: (0, 0)),
        ],
        out_specs=pl.BlockSpec((NC, _B2, 128), lambda i: (0, i, 0)),
        out_shape=jax.ShapeDtypeStruct((NC, NP, 128), jnp.float32),
    )(pacc, u1, dinv, w1, b1)


_B3 = 1000
_G3 = N // _B3


def _tc3_body(pacc_ref, dinv_ref, w_ref, b_ref, wm1_ref, bm1_ref, wm2_ref,
              bm2_ref, out_ref, acc_ref):
    i = pl.program_id(0)
    p = jnp.concatenate([pacc_ref[0], pacc_ref[1]], axis=1) * dinv_ref[...]
    h = _leaky(jnp.dot(p, w_ref[...], preferred_element_type=jnp.float32)
               + b_ref[...])
    part = jnp.sum(h, axis=0, keepdims=True)

    @pl.when(i == 0)
    def _():
        acc_ref[...] = part

    @pl.when(i > 0)
    def _():
        acc_ref[...] += part

    @pl.when(i == _G3 - 1)
    def _():
        g = acc_ref[...] * (1.0 / N)
        m = _leaky(jnp.dot(g, wm1_ref[...], preferred_element_type=jnp.float32)
                   + bm1_ref[...])
        out_ref[...] = _leaky(
            jnp.dot(m, wm2_ref[...], preferred_element_type=jnp.float32)
            + bm2_ref[...])


def _tc3(pacc, dinv, w2, b2, wm1, bm1, wm2, bm2):
    return pl.pallas_call(
        _tc3_body,
        grid=(_G3,),
        in_specs=[
            pl.BlockSpec((NC, _B3, 128), lambda i: (0, i, 0)),
            pl.BlockSpec((_B3, 1), lambda i: (i, 0)),
            pl.BlockSpec((256, 256), lambda i: (0, 0)),
            pl.BlockSpec((1, 256), lambda i: (0, 0)),
            pl.BlockSpec((256, 512), lambda i: (0, 0)),
            pl.BlockSpec((1, 512), lambda i: (0, 0)),
            pl.BlockSpec((512, 128), lambda i: (0, 0)),
            pl.BlockSpec((1, 128), lambda i: (0, 0)),
        ],
        out_specs=pl.BlockSpec((1, 128), lambda i: (0, 0)),
        out_shape=jax.ShapeDtypeStruct((1, 128), jnp.float32),
        scratch_shapes=[pltpu.VMEM((1, 256), jnp.float32)],
    )(pacc, dinv, w2, b2, wm1, bm1, wm2, bm2)


def kernel(x, edge_index, W1, b1, W2, b2, Wm1, bm1, Wm2, bm2):
    src = edge_index[0]
    dst = edge_index[1]
    src_chunks = src.reshape(NCHUNK, CHUNK)
    dst_chunks = dst.reshape(NCHUNK, CHUNK)
    srcg = jnp.stack([src, src + NP]).reshape(NC, NCHUNK, CHUNK)
    xpad = jnp.pad(x, ((0, NP - N), (0, 0)))
    dstd = jnp.stack(
        [jnp.where(dst < NT, dst, NT + (dst & (TR - 1))),
         jnp.where(dst >= NT, dst - NT, NT + (dst & (TR - 1)))],
    ).reshape(NC, NCHUNK, CHUNK)

    deg = _degree(dstd)
    dinv, u1 = _tc1(deg, xpad)

    p1 = _segment_sum1(u1, src_chunks, dstd).reshape(NC, NP, 128)
    u2 = _tc2(p1, u1, dinv, W1, b1.reshape(1, 256))

    u2_flat = u2.reshape(NC * NP, 128)
    p2 = _segment_sum2(u2_flat, srcg, dstd).reshape(NC, NP, 128)
    return _tc3(p2, dinv, W2, b2.reshape(1, 256), Wm1, bm1.reshape(1, 512),
                Wm2, bm2.reshape(1, 128))
